# Initial kernel scaffold; baseline (speedup 1.0000x reference)
import jax
import jax.numpy as jnp
from jax.experimental import pallas as pl


def kernel(X, edge_index, X_edge, D, W_self, b_self, W_edge, W_U, b_U):
    raise NotImplementedError("write your pallas kernel here")


# speedup vs baseline: 1.0005x; 1.0005x over previous
import jax
import jax.numpy as jnp
from jax.experimental import pallas as pl

N = 10000
DELTA = 3.0


def _aggregate(msg, dst, n_nodes):
    ones = jnp.ones((msg.shape[0], 1), dtype=msg.dtype)
    cnt = jax.ops.segment_sum(ones, dst, num_segments=n_nodes)
    s = jax.ops.segment_sum(msg, dst, num_segments=n_nodes)
    mean = s / jnp.maximum(cnt, 1.0)
    mx = jax.ops.segment_max(msg, dst, num_segments=n_nodes)
    mx = jnp.where(cnt > 0, mx, 0.0)
    mean_sq = jax.ops.segment_sum(msg * msg, dst, num_segments=n_nodes) / jnp.maximum(cnt, 1.0)
    var = jax.nn.relu(mean_sq - mean * mean)
    std = jnp.sqrt(var + 1e-20)
    return jnp.concatenate([mean, mx, s, std], axis=1)


def kernel(X, edge_index, X_edge, D, W_self, b_self, W_edge, W_U, b_U):
    src = edge_index[0]
    dst = edge_index[1]
    Xs = X @ W_self + b_self
    msg = Xs[src]
    h = _aggregate(msg, dst, N)
    amp = jnp.log(D.astype(jnp.float32) + 1.0) / DELTA
    h_list = [h, h * amp]
    Xe = X_edge @ W_edge
    he = _aggregate(Xe, dst, N)
    he_list = [he, he * amp]
    hh = jnp.concatenate(h_list + he_list, axis=1)
    out = hh @ W_U + b_U
    return out


# speedup vs baseline: 1.8357x; 1.8348x over previous
import functools

import jax
import jax.numpy as jnp
from jax import lax
from jax.experimental import pallas as pl
from jax.experimental.pallas import tpu as pltpu
from jax.experimental.pallas import tpu_sc as plsc

N = 10000
E = 320000
DF = 128
DE = 16
OUT = 128
DELTA = 3.0

NC = 2
NS = 16
NW = NC * NS
L = 16

NPT = 313
JUNK = NPT
ACC_ROWS = NPT + 1
CNT_W = 320
CHUNK = 1280
NEG_INF = -3.4028235e38


def _mm_bias_body(x_ref, w_ref, b_ref, o_ref):
    o_ref[...] = (
        jnp.dot(x_ref[...], w_ref[...], preferred_element_type=jnp.float32)
        + b_ref[...]
    )


def _tc_project_nodes(X, W, b):
    return pl.pallas_call(
        _mm_bias_body,
        out_shape=jax.ShapeDtypeStruct((N, DF), jnp.float32),
    )(X, W, b.reshape(1, DF))


def _mm_body(x_ref, w_ref, o_ref):
    o_ref[...] = jnp.dot(x_ref[...], w_ref[...],
                         preferred_element_type=jnp.float32)


def _tc_project_edges(X_edge, W_edge):
    blk = 8000
    return pl.pallas_call(
        _mm_body,
        grid=(E // blk,),
        in_specs=[
            pl.BlockSpec((blk, DE), lambda i: (i, 0)),
            pl.BlockSpec((DE, DF), lambda i: (0, 0)),
        ],
        out_specs=pl.BlockSpec((blk, DF), lambda i: (i, 0)),
        out_shape=jax.ShapeDtypeStruct((E, DF), jnp.float32),
    )(X_edge, W_edge)


def _finalize_body(s_a, m_a, q_a, c_a, s_e, m_e, q_e, amp_ref,
                   wid_ref, wamp_ref, b_ref, o_ref):
    cnt = c_a[...]
    cntc = jnp.maximum(cnt, 1.0)
    inv = 1.0 / cntc
    has = cnt > 0.0

    sa = s_a[...]
    mean_a = sa * inv
    mx_a = jnp.where(has, m_a[...], 0.0)
    var_a = jnp.maximum(q_a[...] * inv - mean_a * mean_a, 0.0)
    std_a = jnp.sqrt(var_a + 1e-20)

    se = s_e[...]
    mean_e = se * inv
    mx_e = jnp.where(has, m_e[...], 0.0)
    var_e = jnp.maximum(q_e[...] * inv - mean_e * mean_e, 0.0)
    std_e = jnp.sqrt(var_e + 1e-20)

    hcat = jnp.concatenate(
        [mean_a, mx_a, sa, std_a, mean_e, mx_e, se, std_e], axis=1)
    amp = amp_ref[...]
    o_ref[...] = (
        jnp.dot(hcat, wid_ref[...], preferred_element_type=jnp.float32)
        + amp * jnp.dot(hcat, wamp_ref[...],
                        preferred_element_type=jnp.float32)
        + b_ref[...]
    )


def _tc_finalize(s_a, m_a, q_a, cnt, s_e, m_e, q_e, amp, Wid, Wamp, b_U):
    blk = 1000
    g = N // blk
    node_spec = pl.BlockSpec((blk, DF), lambda i: (i, 0))
    col_spec = pl.BlockSpec((blk, 1), lambda i: (i, 0))
    return pl.pallas_call(
        _finalize_body,
        grid=(g,),
        in_specs=[
            node_spec, node_spec, node_spec, col_spec,
            node_spec, node_spec, node_spec, col_spec,
            pl.BlockSpec((4 * 2 * DF, OUT), lambda i: (0, 0)),
            pl.BlockSpec((4 * 2 * DF, OUT), lambda i: (0, 0)),
            pl.BlockSpec((1, OUT), lambda i: (0, 0)),
        ],
        out_specs=pl.BlockSpec((blk, OUT), lambda i: (i, 0)),
        out_shape=jax.ShapeDtypeStruct((N, OUT), jnp.float32),
    )(s_a, m_a, q_a, cnt, s_e, m_e, q_e, amp, Wid, Wamp, b_U.reshape(1, OUT))


def _acc_init(acc_s, acc_m, acc_q):
    def body(i, _):
        off = pl.ds(i * L, L)
        acc_s[off] = jnp.zeros((L,), jnp.float32)
        acc_m[off] = jnp.full((L,), NEG_INF, jnp.float32)
        acc_q[off] = jnp.zeros((L,), jnp.float32)
        return 0
    lax.fori_loop(0, ACC_ROWS * DF // L, body, 0)


def _accumulate_batches(nb, lidx, loff, rows, sem, table, acc_s, acc_m, acc_q):
    def batch(b, _):
        idx = plsc.Indices(lidx.at[pl.ds(b * L, L)], ignored_value=-1)
        pltpu.async_copy(table.at[idx], rows, sem).wait()
        offv = loff[pl.ds(b * L, L)]
        for j in range(L):
            base = offv[j] * DF
            for fb in range(DF // L):
                r = rows[j, pl.ds(fb * L, L)]
                o = pl.ds(base + fb * L, L)
                acc_s[o] = acc_s[o] + r
                acc_m[o] = jnp.maximum(acc_m[o], r)
                acc_q[o] = acc_q[o] + r * r
        return 0
    lax.fori_loop(0, nb, batch, 0)


def _write_out(wid, acc_s, acc_m, acc_q, out_s, out_m, out_q):
    span = NPT * DF
    pltpu.sync_copy(acc_s.at[pl.ds(0, span)], out_s.at[pl.ds(wid * span, span)])
    pltpu.sync_copy(acc_m.at[pl.ds(0, span)], out_m.at[pl.ds(wid * span, span)])
    pltpu.sync_copy(acc_q.at[pl.ds(0, span)], out_q.at[pl.ds(wid * span, span)])


def _sc_node_pass(Xs, src, dst):
    mesh = plsc.VectorSubcoreMesh(
        core_axis_name="c", subcore_axis_name="s",
        num_cores=NC, num_subcores=NS)

    flat = jax.ShapeDtypeStruct((NW * NPT * DF,), jnp.float32)

    @functools.partial(
        pl.kernel,
        out_type=[flat, flat, flat,
                  jax.ShapeDtypeStruct((NW * CNT_W,), jnp.float32)],
        mesh=mesh,
        compiler_params=pltpu.CompilerParams(needs_layout_passes=False),
        scratch_types=[
            pltpu.VMEM((ACC_ROWS * DF,), jnp.float32),
            pltpu.VMEM((ACC_ROWS * DF,), jnp.float32),
            pltpu.VMEM((ACC_ROWS * DF,), jnp.float32),
            pltpu.VMEM((CNT_W,), jnp.float32),
            pltpu.VMEM((CHUNK,), jnp.int32),
            pltpu.VMEM((CHUNK,), jnp.int32),
            pltpu.VMEM((CHUNK + L,), jnp.int32),
            pltpu.VMEM((CHUNK + L,), jnp.int32),
            pltpu.VMEM((L, DF), jnp.float32),
            pltpu.SemaphoreType.DMA,
        ],
    )
    def kern(xs_hbm, src_hbm, dst_hbm, out_s, out_m, out_q, out_c,
             acc_s, acc_m, acc_q, acc_c, dbuf, sbuf, lsrc, loff, rows,
             sem):
        cid = lax.axis_index("c")
        sid = lax.axis_index("s")
        wid = cid * NS + sid
        lo = wid * NPT

        _acc_init(acc_s, acc_m, acc_q)
        def cbody(i, _):
            acc_c[pl.ds(i * L, L)] = jnp.zeros((L,), jnp.float32)
            return 0
        lax.fori_loop(0, CNT_W // L, cbody, 0)

        ones = jnp.ones((L,), jnp.float32)

        def chunk(c, _):
            base_e = c * CHUNK
            pltpu.sync_copy(dst_hbm.at[pl.ds(base_e, CHUNK)], dbuf)
            pltpu.sync_copy(src_hbm.at[pl.ds(base_e, CHUNK)], sbuf)

            def group(g, k):
                dv = dbuf[pl.ds(g * L, L)]
                sv = sbuf[pl.ds(g * L, L)]
                offv = dv - lo
                m = (offv >= 0) & (offv < NPT)
                pos = k - 1 + plsc.cumsum(m.astype(jnp.int32))
                plsc.store_scatter(lsrc, [pos], sv, mask=m)
                plsc.store_scatter(loff, [pos], offv, mask=m)
                plsc.addupdate_scatter(acc_c, [offv], ones, mask=m)
                return pos[L - 1] + 1

            k = lax.fori_loop(0, CHUNK // L, group, jnp.int32(0))

            lsrc[pl.ds(k, L)] = jnp.full((L,), -1, jnp.int32)
            loff[pl.ds(k, L)] = jnp.full((L,), JUNK, jnp.int32)
            nb = (k + L - 1) // L
            _accumulate_batches(nb, lsrc, loff, rows, sem, xs_hbm,
                                acc_s, acc_m, acc_q)
            return 0

        lax.fori_loop(0, E // CHUNK, chunk, 0)

        _write_out(wid, acc_s, acc_m, acc_q, out_s, out_m, out_q)
        pltpu.sync_copy(acc_c, out_c.at[pl.ds(wid * CNT_W, CNT_W)])

    return kern(Xs, src, dst)


def _sc_edge_pass(Xe, dst):
    mesh = plsc.VectorSubcoreMesh(
        core_axis_name="c", subcore_axis_name="s",
        num_cores=NC, num_subcores=NS)

    flat = jax.ShapeDtypeStruct((NW * NPT * DF,), jnp.float32)

    @functools.partial(
        pl.kernel,
        out_type=[flat, flat, flat],
        mesh=mesh,
        compiler_params=pltpu.CompilerParams(needs_layout_passes=False),
        scratch_types=[
            pltpu.VMEM((ACC_ROWS * DF,), jnp.float32),
            pltpu.VMEM((ACC_ROWS * DF,), jnp.float32),
            pltpu.VMEM((ACC_ROWS * DF,), jnp.float32),
            pltpu.VMEM((CHUNK,), jnp.int32),
            pltpu.VMEM((CHUNK + L,), jnp.int32),
            pltpu.VMEM((CHUNK + L,), jnp.int32),
            pltpu.VMEM((L, DF), jnp.float32),
            pltpu.SemaphoreType.DMA,
        ],
    )
    def kern(xe_hbm, dst_hbm, out_s, out_m, out_q,
             acc_s, acc_m, acc_q, dbuf, lpos, loff, rows, sem):
        cid = lax.axis_index("c")
        sid = lax.axis_index("s")
        wid = cid * NS + sid
        lo = wid * NPT

        _acc_init(acc_s, acc_m, acc_q)
        iota = lax.iota(jnp.int32, L)

        def chunk(c, _):
            base_e = c * CHUNK
            pltpu.sync_copy(dst_hbm.at[pl.ds(base_e, CHUNK)], dbuf)

            def group(g, k):
                dv = dbuf[pl.ds(g * L, L)]
                pv = base_e + g * L + iota
                offv = dv - lo
                m = (offv >= 0) & (offv < NPT)
                pos = k - 1 + plsc.cumsum(m.astype(jnp.int32))
                plsc.store_scatter(lpos, [pos], pv, mask=m)
                plsc.store_scatter(loff, [pos], offv, mask=m)
                return pos[L - 1] + 1

            k = lax.fori_loop(0, CHUNK // L, group, jnp.int32(0))
            lpos[pl.ds(k, L)] = jnp.full((L,), -1, jnp.int32)
            loff[pl.ds(k, L)] = jnp.full((L,), JUNK, jnp.int32)
            nb = (k + L - 1) // L
            _accumulate_batches(nb, lpos, loff, rows, sem, xe_hbm,
                                acc_s, acc_m, acc_q)
            return 0

        lax.fori_loop(0, E // CHUNK, chunk, 0)
        _write_out(wid, acc_s, acc_m, acc_q, out_s, out_m, out_q)

    return kern(Xe, dst)


def kernel(X, edge_index, X_edge, D, W_self, b_self, W_edge, W_U, b_U):
    src = edge_index[0]
    dst = edge_index[1]

    Xs = _tc_project_nodes(X, W_self, b_self)
    Xe = _tc_project_edges(X_edge, W_edge)

    s_a, m_a, q_a, cnt = _sc_node_pass(Xs, src, dst)
    s_e, m_e, q_e = _sc_edge_pass(Xe, dst)

    def rows(v):
        return v.reshape(NW * NPT, DF)[:N]

    cnt_n = cnt.reshape(NW, CNT_W)[:, :NPT].reshape(NW * NPT)[:N, None]
    amp = jnp.log(D.astype(jnp.float32) + 1.0) / DELTA

    H = 4 * DF
    Wid = jnp.concatenate([W_U[0:H], W_U[2 * H:3 * H]], axis=0)
    Wamp = jnp.concatenate([W_U[H:2 * H], W_U[3 * H:4 * H]], axis=0)

    return _tc_finalize(rows(s_a), rows(m_a), rows(q_a), cnt_n,
                        rows(s_e), rows(m_e), rows(q_e), amp,
                        Wid, Wamp, b_U)
